# Initial kernel scaffold; baseline (speedup 1.0000x reference)
import jax
import jax.numpy as jnp
from jax.experimental import pallas as pl


def kernel(x, edge_index, node_type, te_w, te_b, type_emb, proj_w, proj_b, Wl, bl, Wr, br, att, Wres, gbias, ln_g, ln_b, alpha_p, h1_w, h1_b, h2_w, h2_b):
    raise NotImplementedError("write your pallas kernel here")


# speedup vs baseline: 16.4616x; 16.4616x over previous
import functools

import jax
import jax.numpy as jnp
from jax import lax
from jax.experimental import pallas as pl
from jax.experimental.pallas import tpu as pltpu
from jax.experimental.pallas import tpu_sc as plsc

N = 50000
E = 800000
F = 16
H = 64
HEADS = 4
OUTD = 16
LAYERS = 3
NT = 3
TE = 16

NC = 2
NS = 16
NW = NC * NS

K = 128
E1 = E + N
EW_CH = 208
EW = EW_CH * K
E_PAD = NW * EW
ET_CH = 416
ET = ET_CH * K

HALF = 25024
ACC_H = 25600
OUT_H = 25088
NACC = 50176
PAD_DST = 50100

BN = 512
GRID_N = (N + BN - 1) // BN
BE = 1024
GRID_E = E_PAD // BE

HW = 16

_F32 = jnp.float32
_HI = lax.Precision.HIGHEST


def _mesh():
    return plsc.VectorSubcoreMesh(
        core_axis_name="c", subcore_axis_name="s", num_cores=NC,
        num_subcores=NS)


_SC_PARAMS = pltpu.CompilerParams(use_tc_tiling_on_sc=False)


def _sc_gather2(tab1, idx1, tab2, idx2, d1, d2):
    @functools.partial(
        pl.kernel,
        out_type=[jax.ShapeDtypeStruct((E_PAD, d1), _F32),
                  jax.ShapeDtypeStruct((E_PAD, d2), _F32)],
        mesh=_mesh(),
        compiler_params=_SC_PARAMS,
        scratch_types=[pltpu.VMEM((K,), jnp.int32),
                       pltpu.VMEM((K, d1), _F32),
                       pltpu.VMEM((K,), jnp.int32),
                       pltpu.VMEM((K, d2), _F32),
                       pltpu.SemaphoreType.DMA,
                       pltpu.SemaphoreType.DMA],
    )
    def k(t1, i1, t2, i2, o1, o2, iv1, rv1, iv2, rv2, s1, s2):
        c = lax.axis_index("c")
        s = lax.axis_index("s")
        base = (c * NS + s) * EW

        def body(g, carry):
            off = pl.multiple_of(base + g * K, K)
            pltpu.sync_copy(i1.at[pl.ds(off, K)], iv1)
            cp1 = pltpu.async_copy(t1.at[iv1], rv1, s1)
            pltpu.sync_copy(i2.at[pl.ds(off, K)], iv2)
            cp2 = pltpu.async_copy(t2.at[iv2], rv2, s2)
            cp1.wait()
            pltpu.sync_copy(rv1, o1.at[pl.ds(off, K)])
            cp2.wait()
            pltpu.sync_copy(rv2, o2.at[pl.ds(off, K)])
            return carry

        lax.fori_loop(0, EW_CH, body, 0)

    return k(tab1, idx1, tab2, idx2)


def _sc_scatter_denom(ex, dst, z4):
    @functools.partial(
        pl.kernel,
        out_type=[jax.ShapeDtypeStruct((NACC, HW), _F32),
                  jax.ShapeDtypeStruct((NACC, HW), _F32)],
        mesh=_mesh(),
        compiler_params=_SC_PARAMS,
        scratch_types=[pltpu.VMEM_SHARED((NACC, HW), _F32),
                       pltpu.VMEM((K,), jnp.int32),
                       pltpu.VMEM((K, HW), _F32)],
    )
    def k(ex_h, dst_h, z4_h, d0_h, d1_h, shared, iv, rv):
        c = lax.axis_index("c")
        s = lax.axis_index("s")
        for j in range(7):
            b = s + j * NS

            @pl.when(b < NACC // 512)
            def _():
                pltpu.sync_copy(z4_h, shared.at[pl.ds(b * 512, 512)])

        plsc.subcore_barrier()
        base = (c * NS + s) * EW

        def body(g, carry):
            off = pl.multiple_of(base + g * K, K)
            pltpu.sync_copy(dst_h.at[pl.ds(off, K)], iv)
            pltpu.sync_copy(ex_h.at[pl.ds(off, K)], rv)
            pltpu.sync_copy(rv, shared.at[iv], add=True)
            return carry

        lax.fori_loop(0, EW_CH, body, 0)
        plsc.subcore_barrier()
        for j in range(7):
            b = s + j * NS

            @pl.when(b < NACC // 512)
            def _():
                @pl.when(c == 0)
                def _():
                    pltpu.sync_copy(shared.at[pl.ds(b * 512, 512)],
                                    d0_h.at[pl.ds(b * 512, 512)])

                @pl.when(c == 1)
                def _():
                    pltpu.sync_copy(shared.at[pl.ds(b * 512, 512)],
                                    d1_h.at[pl.ds(b * 512, 512)])

    return k(ex, dst, z4)


def _sc_scatter_out(w, dst, z64):
    @functools.partial(
        pl.kernel,
        out_type=[jax.ShapeDtypeStruct((OUT_H, H), _F32),
                  jax.ShapeDtypeStruct((OUT_H, H), _F32)],
        mesh=_mesh(),
        compiler_params=_SC_PARAMS,
        scratch_types=[pltpu.VMEM_SHARED((ACC_H, H), _F32),
                       pltpu.VMEM((K,), jnp.int32),
                       pltpu.VMEM((K,), jnp.int32),
                       pltpu.VMEM((K, H), _F32)],
    )
    def k(w_h, dst_h, z64_h, o0_h, o1_h, shared, iv, lv, rv):
        c = lax.axis_index("c")
        s = lax.axis_index("s")
        for j in range(4):
            b = s + j * NS

            @pl.when(b < ACC_H // 512)
            def _():
                pltpu.sync_copy(z64_h, shared.at[pl.ds(b * 512, 512)])

        plsc.subcore_barrier()
        nbase = c * HALF
        tbase = s * ET

        def body(g, carry):
            off = pl.multiple_of(tbase + g * K, K)
            pltpu.sync_copy(dst_h.at[pl.ds(off, K)], iv)

            def tb(j, carry2):
                p = pl.multiple_of(j * 16, 16)
                v = iv[pl.ds(p, 16)]
                li = v - nbase
                ok = (li >= 0) & (li < HALF)
                lv[pl.ds(p, 16)] = jnp.where(ok, li, HALF)
                return carry2

            lax.fori_loop(0, K // 16, tb, 0)
            pltpu.sync_copy(w_h.at[pl.ds(off, K)], rv)
            pltpu.sync_copy(rv, shared.at[lv], add=True)
            return carry

        lax.fori_loop(0, ET_CH, body, 0)
        plsc.subcore_barrier()
        for j in range(4):
            b = s + j * NS

            @pl.when(b < OUT_H // 512)
            def _():
                @pl.when(c == 0)
                def _():
                    pltpu.sync_copy(shared.at[pl.ds(b * 512, 512)],
                                    o0_h.at[pl.ds(b * 512, 512)])

                @pl.when(c == 1)
                def _():
                    pltpu.sync_copy(shared.at[pl.ds(b * 512, 512)],
                                    o1_h.at[pl.ds(b * 512, 512)])

    return k(w, dst, z64)


def _node_spec(d):
    return pl.BlockSpec((BN, d), lambda i: (i, 0))


def _full_spec(shape):
    return pl.BlockSpec(shape, lambda i: tuple(0 for _ in shape))


def _edge_spec(d):
    return pl.BlockSpec((BE, d), lambda i: (i, 0))


def _tc_prologue(x, nt2, te_w, te_b, type_emb, proj_w, proj_b):
    def body(x_r, nt_r, tew_r, teb_r, temb_r, pw_r, pb_r, out_r):
        h1 = jnp.maximum(
            jnp.dot(x_r[...], tew_r[...], preferred_element_type=_F32,
                    precision=_HI) + teb_r[...], 0.0)
        oh = (nt_r[...] == lax.broadcasted_iota(jnp.int32, (BN, NT), 1)
              ).astype(_F32)
        tf = jnp.dot(oh, temb_r[...], preferred_element_type=_F32,
                     precision=_HI)
        pw = pw_r[...]
        out_r[...] = (jnp.dot(h1, pw[:H, :], preferred_element_type=_F32,
                              precision=_HI)
                      + jnp.dot(tf, pw[H:, :], preferred_element_type=_F32,
                                precision=_HI) + pb_r[...])

    return pl.pallas_call(
        body,
        grid=(GRID_N,),
        in_specs=[_node_spec(F), _node_spec(1), _full_spec((F, H)),
                  _full_spec((1, H)), _full_spec((NT, TE)),
                  _full_spec((H + TE, H)), _full_spec((1, H))],
        out_specs=_node_spec(H),
        out_shape=jax.ShapeDtypeStruct((N, H), _F32),
    )(x, nt2, te_w, te_b, type_emb, proj_w, proj_b)


def _tc_dense(h, wl, bl, wr, br, wres, gb):
    def body(h_r, wl_r, bl_r, wr_r, br_r, wres_r, gb_r, xl_r, xr_r, res_r):
        hb = h_r[...]
        xl_r[...] = jnp.dot(hb, wl_r[...], preferred_element_type=_F32,
                            precision=_HI) + bl_r[...]
        xr_r[...] = jnp.dot(hb, wr_r[...], preferred_element_type=_F32,
                            precision=_HI) + br_r[...]
        res_r[...] = jnp.dot(hb, wres_r[...], preferred_element_type=_F32,
                             precision=_HI) + gb_r[...]

    return pl.pallas_call(
        body,
        grid=(GRID_N,),
        in_specs=[_node_spec(H)] + [_full_spec((H, H)), _full_spec((1, H))] * 3,
        out_specs=[_node_spec(H)] * 3,
        out_shape=[jax.ShapeDtypeStruct((N, H), _F32)] * 3,
    )(h, wl, bl, wr, br, wres, gb)


def _tc_score(a, b, att_flat):
    def body(a_r, b_r, att_r, s_r, mx_r):
        i = pl.program_id(0)
        m = a_r[...] + b_r[...]
        e = jnp.maximum(m, 0.2 * m)
        p = e * att_r[...]
        g = (lax.broadcasted_iota(jnp.int32, (H, HEADS), 0) // OUTD
             == lax.broadcasted_iota(jnp.int32, (H, HEADS), 1)).astype(_F32)
        sb = jnp.dot(p, g, preferred_element_type=_F32, precision=_HI)
        s_r[...] = sb

        @pl.when(i == 0)
        def _():
            mx_r[...] = jnp.full((1, HEADS), -1e30, _F32)

        mx_r[...] = jnp.maximum(mx_r[...], jnp.max(sb, axis=0, keepdims=True))

    return pl.pallas_call(
        body,
        grid=(GRID_E,),
        in_specs=[_edge_spec(H), _edge_spec(H), _full_spec((1, H))],
        out_specs=[_edge_spec(HEADS), _full_spec((1, HEADS))],
        out_shape=[jax.ShapeDtypeStruct((E_PAD, HEADS), _F32),
                   jax.ShapeDtypeStruct((1, HEADS), _F32)],
    )(a, b, att_flat)


def _tc_ex(s, mx):
    def body(s_r, mx_r, ex_r):
        ex = jnp.exp(s_r[...] - mx_r[...])
        ex_r[...] = jnp.concatenate(
            [ex, jnp.zeros((BE, HW - HEADS), _F32)], axis=-1)

    return pl.pallas_call(
        body,
        grid=(GRID_E,),
        in_specs=[_edge_spec(HEADS), _full_spec((1, HEADS))],
        out_specs=_edge_spec(HW),
        out_shape=jax.ShapeDtypeStruct((E_PAD, HW), _F32),
    )(s, mx)


def _tc_weight(a, s, mx, d0, d1):
    def body(a_r, s_r, mx_r, d0_r, d1_r, w_r):
        ex = jnp.exp(s_r[...] - mx_r[...])
        den = (d0_r[...] + d1_r[...])[:, :HEADS] + 1e-16
        w4 = ex / den
        gt = (lax.broadcasted_iota(jnp.int32, (HEADS, H), 1) // OUTD
              == lax.broadcasted_iota(jnp.int32, (HEADS, H), 0)).astype(_F32)
        w_r[...] = a_r[...] * jnp.dot(w4, gt, preferred_element_type=_F32,
                                      precision=_HI)

    return pl.pallas_call(
        body,
        grid=(GRID_E,),
        in_specs=[_edge_spec(H), _edge_spec(HEADS), _full_spec((1, HEADS)),
                  _edge_spec(HW), _edge_spec(HW)],
        out_specs=_edge_spec(H),
        out_shape=jax.ShapeDtypeStruct((E_PAD, H), _F32),
    )(a, s, mx, d0, d1)


def _tc_combine(agg, res, h0, g, b, alpha):
    def body(agg_r, res_r, h0_r, g_r, b_r, al_r, out_r):
        al = al_r[0, 0]
        hn = al * (agg_r[...] + res_r[...]) + (1.0 - al) * h0_r[...]
        mu = jnp.mean(hn, axis=-1, keepdims=True)
        var = jnp.mean((hn - mu) ** 2, axis=-1, keepdims=True)
        out_r[...] = (hn - mu) / jnp.sqrt(var + 1e-5) * g_r[...] + b_r[...]

    return pl.pallas_call(
        body,
        grid=(GRID_N,),
        in_specs=[_node_spec(H)] * 3 + [_full_spec((1, H)),
                                        _full_spec((1, H)),
                                        _full_spec((1, 1))],
        out_specs=_node_spec(H),
        out_shape=jax.ShapeDtypeStruct((N, H), _F32),
    )(agg, res, h0, g, b, alpha)


def _tc_epilogue(h, w1, b1, w2, b2):
    def body(h_r, w1_r, b1_r, w2_r, b2_r, o_r):
        t = jnp.maximum(
            jnp.dot(h_r[...], w1_r[...], preferred_element_type=_F32,
                    precision=_HI) + b1_r[...], 0.0)
        o_r[...] = jax.nn.sigmoid(
            jnp.dot(t, w2_r[...], preferred_element_type=_F32,
                    precision=_HI) + b2_r[...])

    return pl.pallas_call(
        body,
        grid=(GRID_N,),
        in_specs=[_node_spec(H), _full_spec((H, H // 2)),
                  _full_spec((1, H // 2)), _full_spec((H // 2, 1)),
                  _full_spec((1, 1))],
        out_specs=_node_spec(1),
        out_shape=jax.ShapeDtypeStruct((N, 1), _F32),
    )(h, w1, b1, w2, b2)


def kernel(x, edge_index, node_type, te_w, te_b, type_emb, proj_w, proj_b,
           Wl, bl, Wr, br, att, Wres, gbias, ln_g, ln_b, alpha_p,
           h1_w, h1_b, h2_w, h2_b):
    loops = jnp.arange(N, dtype=edge_index.dtype)
    src = jnp.concatenate([edge_index[0], loops])
    dst = jnp.concatenate([edge_index[1], loops])
    src_p = jnp.concatenate(
        [src, jnp.zeros((E_PAD - E1,), jnp.int32)])
    dst_p = jnp.concatenate(
        [dst, jnp.full((E_PAD - E1,), PAD_DST, jnp.int32)])
    z4 = jnp.zeros((512, HW), _F32)
    z64 = jnp.zeros((512, H), _F32)
    alpha = jnp.reshape(alpha_p, (1, 1)).astype(_F32)

    h = _tc_prologue(x, node_type.reshape(N, 1), te_w, te_b.reshape(1, H),
                     type_emb, proj_w, proj_b.reshape(1, H))
    h0 = h
    for l in range(LAYERS):
        xl, xr, res = _tc_dense(h, Wl[l], bl[l].reshape(1, H), Wr[l],
                                br[l].reshape(1, H), Wres[l],
                                gbias[l].reshape(1, H))
        a, bm = _sc_gather2(xl, src_p, xr, dst_p, H, H)
        s, mx = _tc_score(a, bm, att[l].reshape(1, H))
        ex = _tc_ex(s, mx)
        d_0, d_1 = _sc_scatter_denom(ex, dst_p, z4)
        g0, g1 = _sc_gather2(d_0, dst_p, d_1, dst_p, HW, HW)
        w = _tc_weight(a, s, mx, g0, g1)
        o0, o1 = _sc_scatter_out(w, dst_p, z64)
        agg = jnp.concatenate([o0[:HALF], o1[:HALF]], axis=0)[:N]
        h = _tc_combine(agg, res, h0, ln_g[l].reshape(1, H),
                        ln_b[l].reshape(1, H), alpha)
    return _tc_epilogue(h, h1_w, h1_b.reshape(1, H // 2), h2_w,
                        h2_b.reshape(1, 1))


# speedup vs baseline: 23.3280x; 1.4171x over previous
import functools

import jax
import jax.numpy as jnp
from jax import lax
from jax.experimental import pallas as pl
from jax.experimental.pallas import tpu as pltpu
from jax.experimental.pallas import tpu_sc as plsc

N = 50000
E = 800000
F = 16
H = 64
HEADS = 4
OUTD = 16
LAYERS = 3
NT = 3
TE = 16

NC = 2
NS = 16
NW = NC * NS

K = 128
NB = 4
E1 = E + N
EW_CH = 208
EW = EW_CH * K
E_PAD = NW * EW
ET_CH = 416
ET = ET_CH * K

HALF = 25024
ACC_H = 25088
OUT_H = 25088
NBO = 2
NACC = 50176
PAD_DST = 50100

BN = 2048
GRID_N = (N + BN - 1) // BN
BE = 8192
GRID_E = E_PAD // BE

HW = 16

_F32 = jnp.float32
_HI = lax.Precision.HIGHEST


def _mesh():
    return plsc.VectorSubcoreMesh(
        core_axis_name="c", subcore_axis_name="s", num_cores=NC,
        num_subcores=NS)


_SC_PARAMS = pltpu.CompilerParams(use_tc_tiling_on_sc=False)


def _sc_gather2(tab1, idx1, tab2, idx2, d):
    CH = EW_CH

    @functools.partial(
        pl.kernel,
        out_type=[jax.ShapeDtypeStruct((E_PAD, d), _F32),
                  jax.ShapeDtypeStruct((E_PAD, d), _F32)],
        mesh=_mesh(),
        compiler_params=_SC_PARAMS,
        scratch_types=[pltpu.VMEM((CH, K), jnp.int32),
                       pltpu.VMEM((CH, K), jnp.int32),
                       pltpu.VMEM((NB, K, d), _F32)]
                      + [pltpu.SemaphoreType.DMA] * (2 * NB),
    )
    def k(t1, i1, t2, i2, o1, o2, ib1, ib2, rv, *sems):
        gsem = sems[:NB]
        wsem = sems[NB:]
        c = lax.axis_index("c")
        s = lax.axis_index("s")
        wid = c * NS + s
        pltpu.sync_copy(i1.at[pl.ds(wid * CH, CH)], ib1)
        pltpu.sync_copy(i2.at[pl.ds(wid * CH, CH)], ib2)
        for ti, (tab, ib, o) in enumerate(((t1, ib1, o1), (t2, ib2, o2))):
            def outer(g4, carry, tab=tab, ib=ib, o=o, ti=ti):
                descs = []
                for b in range(NB):
                    ch = g4 * NB + b

                    @pl.when((g4 > 0) | (ti > 0))
                    def _(b=b, o=o):
                        pltpu.make_async_copy(
                            rv.at[b], o.at[pl.ds(b * K, K)], wsem[b]).wait()

                    descs.append(
                        pltpu.async_copy(tab.at[ib.at[ch]], rv.at[b],
                                         gsem[b]))
                for b in range(NB):
                    ch = g4 * NB + b
                    off = wid * EW + ch * K
                    descs[b].wait()
                    pltpu.async_copy(rv.at[b], o.at[pl.ds(off, K)], wsem[b])
                return carry

            lax.fori_loop(0, CH // NB, outer, 0)
        for b in range(NB):
            pltpu.make_async_copy(rv.at[b], o2.at[pl.ds(b * K, K)],
                                  wsem[b]).wait()

    return k(tab1, idx1, tab2, idx2)


def _sc_scatter_denom(ex, dst, z4):
    @functools.partial(
        pl.kernel,
        out_type=[jax.ShapeDtypeStruct((NACC, HW), _F32),
                  jax.ShapeDtypeStruct((NACC, HW), _F32)],
        mesh=_mesh(),
        compiler_params=_SC_PARAMS,
        scratch_types=[pltpu.VMEM_SHARED((NACC, HW), _F32),
                       pltpu.VMEM((EW_CH, K), jnp.int32),
                       pltpu.VMEM((NB, K, HW), _F32)]
                      + [pltpu.SemaphoreType.DMA] * NB,
    )
    def k(ex_h, dst_h, z4_h, d0_h, d1_h, shared, ib, rv, *sems):
        c = lax.axis_index("c")
        s = lax.axis_index("s")
        for j in range(7):
            b = s + j * NS

            @pl.when(b < NACC // 512)
            def _():
                pltpu.sync_copy(z4_h, shared.at[pl.ds(b * 512, 512)])

        plsc.subcore_barrier()
        wid = c * NS + s
        pltpu.sync_copy(dst_h.at[pl.ds(wid * EW_CH, EW_CH)], ib)

        def outer(g4, carry):
            descs = []
            for b in range(NB):
                ch = g4 * NB + b
                off = wid * EW + ch * K
                descs.append(
                    pltpu.async_copy(ex_h.at[pl.ds(off, K)], rv.at[b],
                                     sems[b]))
            for b in range(NB):
                ch = g4 * NB + b
                descs[b].wait()
                pltpu.sync_copy(rv.at[b], shared.at[ib.at[ch]], add=True)
            return carry

        lax.fori_loop(0, EW_CH // NB, outer, 0)
        plsc.subcore_barrier()
        for j in range(7):
            b = s + j * NS

            @pl.when(b < NACC // 512)
            def _():
                @pl.when(c == 0)
                def _():
                    pltpu.sync_copy(shared.at[pl.ds(b * 512, 512)],
                                    d0_h.at[pl.ds(b * 512, 512)])

                @pl.when(c == 1)
                def _():
                    pltpu.sync_copy(shared.at[pl.ds(b * 512, 512)],
                                    d1_h.at[pl.ds(b * 512, 512)])

    return k(ex, dst, z4)


def _sc_scatter_out(w, dst, z64):
    @functools.partial(
        pl.kernel,
        out_type=[jax.ShapeDtypeStruct((OUT_H, H), _F32),
                  jax.ShapeDtypeStruct((OUT_H, H), _F32)],
        mesh=_mesh(),
        compiler_params=_SC_PARAMS,
        scratch_types=[pltpu.VMEM_SHARED((ACC_H, H), _F32),
                       pltpu.VMEM((NBO, K), jnp.int32),
                       pltpu.VMEM((NBO, K, H), _F32)]
                      + [pltpu.SemaphoreType.DMA] * (2 * NBO),
    )
    def k(w_h, dst_h, z64_h, o0_h, o1_h, shared, ib, rv, *sems):
        isem = sems[:NBO]
        wsem = sems[NBO:]
        c = lax.axis_index("c")
        s = lax.axis_index("s")
        for j in range(4):
            b = s + j * NS

            @pl.when(b < ACC_H // 512)
            def _():
                pltpu.sync_copy(z64_h, shared.at[pl.ds(b * 512, 512)])

        plsc.subcore_barrier()
        nbase = c * HALF

        def outer(gg, carry):
            idescs = []
            wdescs = []
            for b in range(NBO):
                ch = gg * NBO + b
                off = s * ET + ch * K
                idescs.append(
                    pltpu.async_copy(dst_h.at[pl.ds(s * ET_CH + ch, 1)],
                                     ib.at[pl.ds(b, 1)], isem[b]))
                wdescs.append(
                    pltpu.async_copy(w_h.at[pl.ds(off, K)], rv.at[b],
                                     wsem[b]))
            for b in range(NBO):
                idescs[b].wait()
                for j in range(K // 16):
                    v = ib[b, pl.ds(j * 16, 16)]
                    li = v - nbase
                    ok = (li >= 0) & (li < HALF)
                    ib[b, pl.ds(j * 16, 16)] = jnp.where(ok, li, HALF)
                wdescs[b].wait()
                pltpu.sync_copy(rv.at[b], shared.at[ib.at[b]], add=True)
            return carry

        lax.fori_loop(0, ET_CH // NBO, outer, 0)
        plsc.subcore_barrier()
        for j in range(4):
            b = s + j * NS

            @pl.when(b < OUT_H // 512)
            def _():
                @pl.when(c == 0)
                def _():
                    pltpu.sync_copy(shared.at[pl.ds(b * 512, 512)],
                                    o0_h.at[pl.ds(b * 512, 512)])

                @pl.when(c == 1)
                def _():
                    pltpu.sync_copy(shared.at[pl.ds(b * 512, 512)],
                                    o1_h.at[pl.ds(b * 512, 512)])

    return k(w, dst, z64)


def _node_spec(d):
    return pl.BlockSpec((BN, d), lambda i: (i, 0))


def _full_spec(shape):
    return pl.BlockSpec(shape, lambda i: tuple(0 for _ in shape))


def _edge_spec(d):
    return pl.BlockSpec((BE, d), lambda i: (i, 0))


def _tc_prologue(x, nt2, te_w, te_b, type_emb, proj_w, proj_b):
    def body(x_r, nt_r, tew_r, teb_r, temb_r, pw_r, pb_r, out_r):
        h1 = jnp.maximum(
            jnp.dot(x_r[...], tew_r[...], preferred_element_type=_F32,
                    precision=_HI) + teb_r[...], 0.0)
        oh = (nt_r[...] == lax.broadcasted_iota(jnp.int32, (BN, NT), 1)
              ).astype(_F32)
        tf = jnp.dot(oh, temb_r[...], preferred_element_type=_F32,
                     precision=_HI)
        pw = pw_r[...]
        out_r[...] = (jnp.dot(h1, pw[:H, :], preferred_element_type=_F32,
                              precision=_HI)
                      + jnp.dot(tf, pw[H:, :], preferred_element_type=_F32,
                                precision=_HI) + pb_r[...])

    return pl.pallas_call(
        body,
        grid=(GRID_N,),
        in_specs=[_node_spec(F), _node_spec(1), _full_spec((F, H)),
                  _full_spec((1, H)), _full_spec((NT, TE)),
                  _full_spec((H + TE, H)), _full_spec((1, H))],
        out_specs=_node_spec(H),
        out_shape=jax.ShapeDtypeStruct((N, H), _F32),
    )(x, nt2, te_w, te_b, type_emb, proj_w, proj_b)


def _tc_dense(h, wl, bl, wr, br, wres, gb):
    def body(h_r, wl_r, bl_r, wr_r, br_r, wres_r, gb_r, xl_r, xr_r, res_r):
        hb = h_r[...]
        xl_r[...] = jnp.dot(hb, wl_r[...], preferred_element_type=_F32,
                            precision=_HI) + bl_r[...]
        xr_r[...] = jnp.dot(hb, wr_r[...], preferred_element_type=_F32,
                            precision=_HI) + br_r[...]
        res_r[...] = jnp.dot(hb, wres_r[...], preferred_element_type=_F32,
                             precision=_HI) + gb_r[...]

    return pl.pallas_call(
        body,
        grid=(GRID_N,),
        in_specs=[_node_spec(H)] + [_full_spec((H, H)), _full_spec((1, H))] * 3,
        out_specs=[_node_spec(H)] * 3,
        out_shape=[jax.ShapeDtypeStruct((N, H), _F32)] * 3,
    )(h, wl, bl, wr, br, wres, gb)


def _tc_score(a, b, att_flat):
    def body(a_r, b_r, att_r, s_r, mx_r):
        i = pl.program_id(0)
        m = a_r[...] + b_r[...]
        e = jnp.maximum(m, 0.2 * m)
        p = e * att_r[...]
        g = (lax.broadcasted_iota(jnp.int32, (H, HEADS), 0) // OUTD
             == lax.broadcasted_iota(jnp.int32, (H, HEADS), 1)).astype(_F32)
        sb = jnp.dot(p, g, preferred_element_type=_F32, precision=_HI)
        s_r[...] = sb

        @pl.when(i == 0)
        def _():
            mx_r[...] = jnp.full((1, HEADS), -1e30, _F32)

        mx_r[...] = jnp.maximum(mx_r[...], jnp.max(sb, axis=0, keepdims=True))

    return pl.pallas_call(
        body,
        grid=(GRID_E,),
        in_specs=[_edge_spec(H), _edge_spec(H), _full_spec((1, H))],
        out_specs=[_edge_spec(HEADS), _full_spec((1, HEADS))],
        out_shape=[jax.ShapeDtypeStruct((E_PAD, HEADS), _F32),
                   jax.ShapeDtypeStruct((1, HEADS), _F32)],
    )(a, b, att_flat)


def _tc_ex(s, mx):
    def body(s_r, mx_r, ex_r):
        ex = jnp.exp(s_r[...] - mx_r[...])
        ex_r[...] = jnp.concatenate(
            [ex, jnp.zeros((BE, HW - HEADS), _F32)], axis=-1)

    return pl.pallas_call(
        body,
        grid=(GRID_E,),
        in_specs=[_edge_spec(HEADS), _full_spec((1, HEADS))],
        out_specs=_edge_spec(HW),
        out_shape=jax.ShapeDtypeStruct((E_PAD, HW), _F32),
    )(s, mx)


def _tc_weight(a, s, mx, d0, d1):
    def body(a_r, s_r, mx_r, d0_r, d1_r, w_r):
        ex = jnp.exp(s_r[...] - mx_r[...])
        den = (d0_r[...] + d1_r[...])[:, :HEADS] + 1e-16
        w4 = ex / den
        gt = (lax.broadcasted_iota(jnp.int32, (HEADS, H), 1) // OUTD
              == lax.broadcasted_iota(jnp.int32, (HEADS, H), 0)).astype(_F32)
        w_r[...] = a_r[...] * jnp.dot(w4, gt, preferred_element_type=_F32,
                                      precision=_HI)

    return pl.pallas_call(
        body,
        grid=(GRID_E,),
        in_specs=[_edge_spec(H), _edge_spec(HEADS), _full_spec((1, HEADS)),
                  _edge_spec(HW), _edge_spec(HW)],
        out_specs=_edge_spec(H),
        out_shape=jax.ShapeDtypeStruct((E_PAD, H), _F32),
    )(a, s, mx, d0, d1)


def _tc_combine(agg, res, h0, g, b, alpha):
    def body(agg_r, res_r, h0_r, g_r, b_r, al_r, out_r):
        al = al_r[0, 0]
        hn = al * (agg_r[...] + res_r[...]) + (1.0 - al) * h0_r[...]
        mu = jnp.mean(hn, axis=-1, keepdims=True)
        var = jnp.mean((hn - mu) ** 2, axis=-1, keepdims=True)
        out_r[...] = (hn - mu) / jnp.sqrt(var + 1e-5) * g_r[...] + b_r[...]

    return pl.pallas_call(
        body,
        grid=(GRID_N,),
        in_specs=[_node_spec(H)] * 3 + [_full_spec((1, H)),
                                        _full_spec((1, H)),
                                        _full_spec((1, 1))],
        out_specs=_node_spec(H),
        out_shape=jax.ShapeDtypeStruct((N, H), _F32),
    )(agg, res, h0, g, b, alpha)


def _tc_epilogue(h, w1, b1, w2, b2):
    def body(h_r, w1_r, b1_r, w2_r, b2_r, o_r):
        t = jnp.maximum(
            jnp.dot(h_r[...], w1_r[...], preferred_element_type=_F32,
                    precision=_HI) + b1_r[...], 0.0)
        o_r[...] = jax.nn.sigmoid(
            jnp.dot(t, w2_r[...], preferred_element_type=_F32,
                    precision=_HI) + b2_r[...])

    return pl.pallas_call(
        body,
        grid=(GRID_N,),
        in_specs=[_node_spec(H), _full_spec((H, H // 2)),
                  _full_spec((1, H // 2)), _full_spec((H // 2, 1)),
                  _full_spec((1, 1))],
        out_specs=_node_spec(1),
        out_shape=jax.ShapeDtypeStruct((N, 1), _F32),
    )(h, w1, b1, w2, b2)


def kernel(x, edge_index, node_type, te_w, te_b, type_emb, proj_w, proj_b,
           Wl, bl, Wr, br, att, Wres, gbias, ln_g, ln_b, alpha_p,
           h1_w, h1_b, h2_w, h2_b):
    loops = jnp.arange(N, dtype=edge_index.dtype)
    src = jnp.concatenate([edge_index[0], loops])
    dst = jnp.concatenate([edge_index[1], loops])
    src_p = jnp.concatenate(
        [src, jnp.zeros((E_PAD - E1,), jnp.int32)]).reshape(E_PAD // K, K)
    dst_p = jnp.concatenate(
        [dst, jnp.full((E_PAD - E1,), PAD_DST, jnp.int32)]).reshape(
            E_PAD // K, K)
    z4 = jnp.zeros((512, HW), _F32)
    z64 = jnp.zeros((512, H), _F32)
    alpha = jnp.reshape(alpha_p, (1, 1)).astype(_F32)

    h = _tc_prologue(x, node_type.reshape(N, 1), te_w, te_b.reshape(1, H),
                     type_emb, proj_w, proj_b.reshape(1, H))
    h0 = h
    for l in range(LAYERS):
        xl, xr, res = _tc_dense(h, Wl[l], bl[l].reshape(1, H), Wr[l],
                                br[l].reshape(1, H), Wres[l],
                                gbias[l].reshape(1, H))
        a, bm = _sc_gather2(xl, src_p, xr, dst_p, H)
        s, mx = _tc_score(a, bm, att[l].reshape(1, H))
        ex = _tc_ex(s, mx)
        d_0, d_1 = _sc_scatter_denom(ex, dst_p, z4)
        g0, g1 = _sc_gather2(d_0, dst_p, d_1, dst_p, HW)
        w = _tc_weight(a, s, mx, g0, g1)
        o0, o1 = _sc_scatter_out(w, dst_p, z64)
        agg = jnp.concatenate([o0[:HALF], o1[:HALF]], axis=0)[:N]
        h = _tc_combine(agg, res, h0, ln_g[l].reshape(1, H),
                        ln_b[l].reshape(1, H), alpha)
    return _tc_epilogue(h, h1_w, h1_b.reshape(1, H // 2), h2_w,
                        h2_b.reshape(1, 1))


# speedup vs baseline: 31.7946x; 1.3629x over previous
import functools

import jax
import jax.numpy as jnp
from jax import lax
from jax.experimental import pallas as pl
from jax.experimental.pallas import tpu as pltpu
from jax.experimental.pallas import tpu_sc as plsc

N = 50000
E = 800000
F = 16
H = 64
HEADS = 4
OUTD = 16
LAYERS = 3
NT = 3
TE = 16

NC = 2
NS = 16
NW = NC * NS

K = 128
NB = 4
E1 = E + N
EW_CH = 208
EW = EW_CH * K
E_PAD = NW * EW
ET_CH = 416
ET = ET_CH * K

HALF = 25024
ACC_H = 25088
OUT_H = 25088
NBO = 2
NACC = 50176
PAD_DST = 50100

BN = 2048
GRID_N = (N + BN - 1) // BN
BE = 8192
GRID_E = E_PAD // BE

HW = 16

_F32 = jnp.float32
_HI = lax.Precision.HIGHEST


def _mesh():
    return plsc.VectorSubcoreMesh(
        core_axis_name="c", subcore_axis_name="s", num_cores=NC,
        num_subcores=NS)


_SC_PARAMS = pltpu.CompilerParams(use_tc_tiling_on_sc=False)


def _sc_gather2(tab1, idx1, tab2, idx2, d):
    CH = EW_CH

    @functools.partial(
        pl.kernel,
        out_type=[jax.ShapeDtypeStruct((E_PAD, d), _F32),
                  jax.ShapeDtypeStruct((E_PAD, d), _F32)],
        mesh=_mesh(),
        compiler_params=_SC_PARAMS,
        scratch_types=[pltpu.VMEM((CH, K), jnp.int32),
                       pltpu.VMEM((CH, K), jnp.int32),
                       pltpu.VMEM((NB, K, d), _F32)]
                      + [pltpu.SemaphoreType.DMA] * (2 * NB),
    )
    def k(t1, i1, t2, i2, o1, o2, ib1, ib2, rv, *sems):
        gsem = sems[:NB]
        wsem = sems[NB:]
        c = lax.axis_index("c")
        s = lax.axis_index("s")
        wid = c * NS + s
        pltpu.sync_copy(i1.at[pl.ds(wid * CH, CH)], ib1)
        pltpu.sync_copy(i2.at[pl.ds(wid * CH, CH)], ib2)
        for ti, (tab, ib, o) in enumerate(((t1, ib1, o1), (t2, ib2, o2))):
            def outer(g4, carry, tab=tab, ib=ib, o=o, ti=ti):
                descs = []
                for b in range(NB):
                    ch = g4 * NB + b

                    @pl.when((g4 > 0) | (ti > 0))
                    def _(b=b, o=o):
                        pltpu.make_async_copy(
                            rv.at[b], o.at[pl.ds(b * K, K)], wsem[b]).wait()

                    descs.append(
                        pltpu.async_copy(tab.at[ib.at[ch]], rv.at[b],
                                         gsem[b]))
                for b in range(NB):
                    ch = g4 * NB + b
                    off = wid * EW + ch * K
                    descs[b].wait()
                    pltpu.async_copy(rv.at[b], o.at[pl.ds(off, K)], wsem[b])
                return carry

            lax.fori_loop(0, CH // NB, outer, 0)
        for b in range(NB):
            pltpu.make_async_copy(rv.at[b], o2.at[pl.ds(b * K, K)],
                                  wsem[b]).wait()

    return k(tab1, idx1, tab2, idx2)


def _sc_scatter_denom(ex, dst, z4):
    @functools.partial(
        pl.kernel,
        out_type=[jax.ShapeDtypeStruct((NACC, HW), _F32),
                  jax.ShapeDtypeStruct((NACC, HW), _F32)],
        mesh=_mesh(),
        compiler_params=_SC_PARAMS,
        scratch_types=[pltpu.VMEM_SHARED((NACC, HW), _F32),
                       pltpu.VMEM((EW_CH, K), jnp.int32),
                       pltpu.VMEM((NB, K, HW), _F32)]
                      + [pltpu.SemaphoreType.DMA] * NB,
    )
    def k(ex_h, dst_h, z4_h, d0_h, d1_h, shared, ib, rv, *sems):
        c = lax.axis_index("c")
        s = lax.axis_index("s")
        for j in range(7):
            b = s + j * NS

            @pl.when(b < NACC // 512)
            def _():
                pltpu.sync_copy(z4_h, shared.at[pl.ds(b * 512, 512)])

        plsc.subcore_barrier()
        wid = c * NS + s
        pltpu.sync_copy(dst_h.at[pl.ds(wid * EW_CH, EW_CH)], ib)

        def outer(g4, carry):
            descs = []
            for b in range(NB):
                ch = g4 * NB + b
                off = wid * EW + ch * K
                descs.append(
                    pltpu.async_copy(ex_h.at[pl.ds(off, K)], rv.at[b],
                                     sems[b]))
            for b in range(NB):
                ch = g4 * NB + b
                descs[b].wait()
                pltpu.sync_copy(rv.at[b], shared.at[ib.at[ch]], add=True)
            return carry

        lax.fori_loop(0, EW_CH // NB, outer, 0)
        plsc.subcore_barrier()
        for j in range(7):
            b = s + j * NS

            @pl.when(b < NACC // 512)
            def _():
                @pl.when(c == 0)
                def _():
                    pltpu.sync_copy(shared.at[pl.ds(b * 512, 512)],
                                    d0_h.at[pl.ds(b * 512, 512)])

                @pl.when(c == 1)
                def _():
                    pltpu.sync_copy(shared.at[pl.ds(b * 512, 512)],
                                    d1_h.at[pl.ds(b * 512, 512)])

    return k(ex, dst, z4)


def _sc_scatter_out(w, dst, z64):
    @functools.partial(
        pl.kernel,
        out_type=[jax.ShapeDtypeStruct((OUT_H, H), _F32),
                  jax.ShapeDtypeStruct((OUT_H, H), _F32)],
        mesh=_mesh(),
        compiler_params=_SC_PARAMS,
        scratch_types=[pltpu.VMEM_SHARED((ACC_H, H), _F32),
                       pltpu.VMEM((NBO, K), jnp.int32),
                       pltpu.VMEM((NBO, K, H), _F32)]
                      + [pltpu.SemaphoreType.DMA] * (2 * NBO),
    )
    def k(w_h, dst_h, z64_h, o0_h, o1_h, shared, ib, rv, *sems):
        isem = sems[:NBO]
        wsem = sems[NBO:]
        c = lax.axis_index("c")
        s = lax.axis_index("s")
        for j in range(4):
            b = s + j * NS

            @pl.when(b < ACC_H // 512)
            def _():
                pltpu.sync_copy(z64_h, shared.at[pl.ds(b * 512, 512)])

        plsc.subcore_barrier()
        nbase = c * HALF

        def outer(gg, carry):
            idescs = []
            wdescs = []
            for b in range(NBO):
                ch = gg * NBO + b
                off = s * ET + ch * K
                idescs.append(
                    pltpu.async_copy(dst_h.at[pl.ds(s * ET_CH + ch, 1)],
                                     ib.at[pl.ds(b, 1)], isem[b]))
                wdescs.append(
                    pltpu.async_copy(w_h.at[pl.ds(off, K)], rv.at[b],
                                     wsem[b]))
            for b in range(NBO):
                idescs[b].wait()
                for j in range(K // 16):
                    v = ib[b, pl.ds(j * 16, 16)]
                    li = v - nbase
                    ok = (li >= 0) & (li < HALF)
                    ib[b, pl.ds(j * 16, 16)] = jnp.where(ok, li, HALF)
                wdescs[b].wait()
                pltpu.sync_copy(rv.at[b], shared.at[ib.at[b]], add=True)
            return carry

        lax.fori_loop(0, ET_CH // NBO, outer, 0)
        plsc.subcore_barrier()
        for j in range(4):
            b = s + j * NS

            @pl.when(b < OUT_H // 512)
            def _():
                @pl.when(c == 0)
                def _():
                    pltpu.sync_copy(shared.at[pl.ds(b * 512, 512)],
                                    o0_h.at[pl.ds(b * 512, 512)])

                @pl.when(c == 1)
                def _():
                    pltpu.sync_copy(shared.at[pl.ds(b * 512, 512)],
                                    o1_h.at[pl.ds(b * 512, 512)])

    return k(w, dst, z64)


def _node_spec(d):
    return pl.BlockSpec((BN, d), lambda i: (i, 0))


def _full_spec(shape):
    return pl.BlockSpec(shape, lambda i: tuple(0 for _ in shape))


def _edge_spec(d):
    return pl.BlockSpec((BE, d), lambda i: (i, 0))


def _tc_prologue(x, nt2, te_w, te_b, type_emb, proj_w, proj_b):
    def body(x_r, nt_r, tew_r, teb_r, temb_r, pw_r, pb_r, out_r):
        h1 = jnp.maximum(
            jnp.dot(x_r[...], tew_r[...], preferred_element_type=_F32,
                    precision=_HI) + teb_r[...], 0.0)
        oh = (nt_r[...] == lax.broadcasted_iota(jnp.int32, (BN, NT), 1)
              ).astype(_F32)
        tf = jnp.dot(oh, temb_r[...], preferred_element_type=_F32,
                     precision=_HI)
        pw = pw_r[...]
        out_r[...] = (jnp.dot(h1, pw[:H, :], preferred_element_type=_F32,
                              precision=_HI)
                      + jnp.dot(tf, pw[H:, :], preferred_element_type=_F32,
                                precision=_HI) + pb_r[...])

    return pl.pallas_call(
        body,
        grid=(GRID_N,),
        in_specs=[_node_spec(F), _node_spec(1), _full_spec((F, H)),
                  _full_spec((1, H)), _full_spec((NT, TE)),
                  _full_spec((H + TE, H)), _full_spec((1, H))],
        out_specs=_node_spec(H),
        out_shape=jax.ShapeDtypeStruct((N, H), _F32),
    )(x, nt2, te_w, te_b, type_emb, proj_w, proj_b)


def _tc_dense(h, wl, bl, wr, br, wres, gb):
    def body(h_r, wl_r, bl_r, wr_r, br_r, wres_r, gb_r, xl_r, xr_r, res_r):
        hb = h_r[...]
        xl_r[...] = jnp.dot(hb, wl_r[...], preferred_element_type=_F32,
                            precision=_HI) + bl_r[...]
        xr_r[...] = jnp.dot(hb, wr_r[...], preferred_element_type=_F32,
                            precision=_HI) + br_r[...]
        res_r[...] = jnp.dot(hb, wres_r[...], preferred_element_type=_F32,
                             precision=_HI) + gb_r[...]

    return pl.pallas_call(
        body,
        grid=(GRID_N,),
        in_specs=[_node_spec(H)] + [_full_spec((H, H)), _full_spec((1, H))] * 3,
        out_specs=[_node_spec(H)] * 3,
        out_shape=[jax.ShapeDtypeStruct((N, H), _F32)] * 3,
    )(h, wl, bl, wr, br, wres, gb)


def _tc_score(a, b, att_flat):
    def body(a_r, b_r, att_r, s_r, mx_r):
        i = pl.program_id(0)
        m = a_r[...] + b_r[...]
        e = jnp.maximum(m, 0.2 * m)
        p = e * att_r[...]
        g = (lax.broadcasted_iota(jnp.int32, (H, HEADS), 0) // OUTD
             == lax.broadcasted_iota(jnp.int32, (H, HEADS), 1)).astype(_F32)
        sb = jnp.dot(p, g, preferred_element_type=_F32, precision=_HI)
        s_r[...] = sb

        @pl.when(i == 0)
        def _():
            mx_r[...] = jnp.full((1, HEADS), -1e30, _F32)

        mx_r[...] = jnp.maximum(mx_r[...], jnp.max(sb, axis=0, keepdims=True))

    return pl.pallas_call(
        body,
        grid=(GRID_E,),
        in_specs=[_edge_spec(H), _edge_spec(H), _full_spec((1, H))],
        out_specs=[_edge_spec(HEADS), _full_spec((1, HEADS))],
        out_shape=[jax.ShapeDtypeStruct((E_PAD, HEADS), _F32),
                   jax.ShapeDtypeStruct((1, HEADS), _F32)],
    )(a, b, att_flat)


def _tc_ex(s, mx):
    def body(s_r, mx_r, ex_r):
        ex = jnp.exp(s_r[...] - mx_r[...])
        ex_r[...] = jnp.concatenate(
            [ex, jnp.zeros((BE, HW - HEADS), _F32)], axis=-1)

    return pl.pallas_call(
        body,
        grid=(GRID_E,),
        in_specs=[_edge_spec(HEADS), _full_spec((1, HEADS))],
        out_specs=_edge_spec(HW),
        out_shape=jax.ShapeDtypeStruct((E_PAD, HW), _F32),
    )(s, mx)


def _tc_weight(a, s, mx):
    def body(a_r, s_r, mx_r, w_r):
        ex = jnp.exp(s_r[...] - mx_r[...])
        gt = (lax.broadcasted_iota(jnp.int32, (HEADS, H), 1) // OUTD
              == lax.broadcasted_iota(jnp.int32, (HEADS, H), 0)).astype(_F32)
        w_r[...] = a_r[...] * jnp.dot(ex, gt, preferred_element_type=_F32,
                                      precision=_HI)

    return pl.pallas_call(
        body,
        grid=(GRID_E,),
        in_specs=[_edge_spec(H), _edge_spec(HEADS), _full_spec((1, HEADS))],
        out_specs=_edge_spec(H),
        out_shape=jax.ShapeDtypeStruct((E_PAD, H), _F32),
    )(a, s, mx)


def _tc_combine(agg, den0, den1, res, h0, g, b, alpha):
    def body(agg_r, d0_r, d1_r, res_r, h0_r, g_r, b_r, al_r, out_r):
        al = al_r[0, 0]
        den = (d0_r[...] + d1_r[...])[:, :HEADS] + 1e-16
        gt = (lax.broadcasted_iota(jnp.int32, (HEADS, H), 1) // OUTD
              == lax.broadcasted_iota(jnp.int32, (HEADS, H), 0)).astype(_F32)
        den64 = jnp.dot(den, gt, preferred_element_type=_F32, precision=_HI)
        hn = al * (agg_r[...] / den64 + res_r[...]) + (1.0 - al) * h0_r[...]
        mu = jnp.mean(hn, axis=-1, keepdims=True)
        var = jnp.mean((hn - mu) ** 2, axis=-1, keepdims=True)
        out_r[...] = (hn - mu) / jnp.sqrt(var + 1e-5) * g_r[...] + b_r[...]

    return pl.pallas_call(
        body,
        grid=(GRID_N,),
        in_specs=[_node_spec(H), _node_spec(HW), _node_spec(HW),
                  _node_spec(H), _node_spec(H), _full_spec((1, H)),
                  _full_spec((1, H)), _full_spec((1, 1))],
        out_specs=_node_spec(H),
        out_shape=jax.ShapeDtypeStruct((N, H), _F32),
    )(agg, den0, den1, res, h0, g, b, alpha)


def _tc_epilogue(h, w1, b1, w2, b2):
    def body(h_r, w1_r, b1_r, w2_r, b2_r, o_r):
        t = jnp.maximum(
            jnp.dot(h_r[...], w1_r[...], preferred_element_type=_F32,
                    precision=_HI) + b1_r[...], 0.0)
        o_r[...] = jax.nn.sigmoid(
            jnp.dot(t, w2_r[...], preferred_element_type=_F32,
                    precision=_HI) + b2_r[...])

    return pl.pallas_call(
        body,
        grid=(GRID_N,),
        in_specs=[_node_spec(H), _full_spec((H, H // 2)),
                  _full_spec((1, H // 2)), _full_spec((H // 2, 1)),
                  _full_spec((1, 1))],
        out_specs=_node_spec(1),
        out_shape=jax.ShapeDtypeStruct((N, 1), _F32),
    )(h, w1, b1, w2, b2)


def kernel(x, edge_index, node_type, te_w, te_b, type_emb, proj_w, proj_b,
           Wl, bl, Wr, br, att, Wres, gbias, ln_g, ln_b, alpha_p,
           h1_w, h1_b, h2_w, h2_b):
    loops = jnp.arange(N, dtype=edge_index.dtype)
    src = jnp.concatenate([edge_index[0], loops])
    dst = jnp.concatenate([edge_index[1], loops])
    src_p = jnp.concatenate(
        [src, jnp.zeros((E_PAD - E1,), jnp.int32)]).reshape(E_PAD // K, K)
    dst_p = jnp.concatenate(
        [dst, jnp.full((E_PAD - E1,), PAD_DST, jnp.int32)]).reshape(
            E_PAD // K, K)
    z4 = jnp.zeros((512, HW), _F32)
    z64 = jnp.zeros((512, H), _F32)
    alpha = jnp.reshape(alpha_p, (1, 1)).astype(_F32)

    h = _tc_prologue(x, node_type.reshape(N, 1), te_w, te_b.reshape(1, H),
                     type_emb, proj_w, proj_b.reshape(1, H))
    h0 = h
    for l in range(LAYERS):
        xl, xr, res = _tc_dense(h, Wl[l], bl[l].reshape(1, H), Wr[l],
                                br[l].reshape(1, H), Wres[l],
                                gbias[l].reshape(1, H))
        a, bm = _sc_gather2(xl, src_p, xr, dst_p, H)
        s, mx = _tc_score(a, bm, att[l].reshape(1, H))
        ex = _tc_ex(s, mx)
        d_0, d_1 = _sc_scatter_denom(ex, dst_p, z4)
        w = _tc_weight(a, s, mx)
        o0, o1 = _sc_scatter_out(w, dst_p, z64)
        agg = jnp.concatenate([o0[:HALF], o1[:HALF]], axis=0)[:N]
        h = _tc_combine(agg, d_0[:N], d_1[:N], res, h0,
                        ln_g[l].reshape(1, H), ln_b[l].reshape(1, H), alpha)
    return _tc_epilogue(h, h1_w, h1_b.reshape(1, H // 2), h2_w,
                        h2_b.reshape(1, 1))
